# Initial kernel scaffold; baseline (speedup 1.0000x reference)
import jax
import jax.numpy as jnp
from jax.experimental import pallas as pl


def kernel(q_sub, q_rel, hidden, edges, n_node, old_nodes_new_idx, rela_embed, Ws, Wr, Wqr, bqr, Wa, Wh):
    raise NotImplementedError("write your pallas kernel here")


# speedup vs baseline: 2.8600x; 2.8600x over previous
import functools
import jax
import jax.numpy as jnp
from jax import lax
from jax.experimental import pallas as pl
from jax.experimental.pallas import tpu as pltpu
from jax.experimental.pallas import tpu_sc as plsc

MN = 1e-15
MC = 1e-06
BALL_EPS = 0.004


def _sc_gather(table, idx, K):
    M = idx.shape[0]
    C = table.shape[1]
    info = plsc.get_sparse_core_info()
    NC, NS = info.num_cores, info.num_subcores
    NW = NC * NS
    per_w = M // NW
    steps = per_w // K
    mesh = plsc.VectorSubcoreMesh(core_axis_name="c", subcore_axis_name="s")

    @functools.partial(
        pl.kernel, mesh=mesh,
        out_type=jax.ShapeDtypeStruct((M, C), jnp.float32),
        scratch_types=[
            pltpu.VMEM((K,), jnp.int32),
            pltpu.VMEM((K, C), jnp.float32),
            pltpu.SemaphoreType.DMA,
        ],
    )
    def gk(table_hbm, idx_hbm, out_hbm, idx_v, rows_v, sem):
        wid = lax.axis_index("s") * NC + lax.axis_index("c")
        base = wid * per_w

        def step(i, c):
            off = base + i * K
            pltpu.sync_copy(idx_hbm.at[pl.ds(off, K)], idx_v)
            pltpu.async_copy(table_hbm.at[idx_v], rows_v, sem).wait()
            pltpu.sync_copy(rows_v, out_hbm.at[pl.ds(off, K)])
            return c

        lax.fori_loop(0, steps, step, 0)

    return gk(table, idx)


def _sc_scatter_add(msg, obj, half, K):
    E, D = msg.shape
    info = plsc.get_sparse_core_info()
    NC, NS = info.num_cores, info.num_subcores
    per_s = E // NS
    steps = per_s // K
    AR = half + 128
    zrow = AR // NS
    drow = half // NS
    zeros = jnp.zeros((AR, D), jnp.float32)
    mesh = plsc.VectorSubcoreMesh(core_axis_name="c", subcore_axis_name="s")

    @functools.partial(
        pl.kernel, mesh=mesh,
        out_type=jax.ShapeDtypeStruct((NC * half, D), jnp.float32),
        scratch_types=[
            pltpu.VMEM((K,), jnp.int32),
            pltpu.VMEM((K, D), jnp.float32),
            pltpu.VMEM_SHARED((AR, D), jnp.float32),
        ],
    )
    def sk(msg_hbm, obj_hbm, z_hbm, out_hbm, idx_v, rows_v, acc):
        cid = lax.axis_index("c")
        sid = lax.axis_index("s")
        z0 = sid * zrow
        pltpu.sync_copy(z_hbm.at[pl.ds(z0, zrow)], acc.at[pl.ds(z0, zrow)])
        plsc.subcore_barrier()
        lo = cid * half
        base = sid * per_s

        def step(i, c):
            off = base + i * K
            pltpu.sync_copy(obj_hbm.at[pl.ds(off, K)], idx_v)
            pltpu.sync_copy(msg_hbm.at[pl.ds(off, K)], rows_v)

            def remap(j, c2):
                v = idx_v[pl.ds(j * 16, 16)]
                loc = v - lo
                ok = (loc >= 0) & (loc < half)
                idx_v[pl.ds(j * 16, 16)] = jnp.where(ok, loc, half)
                return c2

            lax.fori_loop(0, K // 16, remap, 0)
            pltpu.sync_copy(rows_v, acc.at[idx_v], add=True)
            return c

        lax.fori_loop(0, steps, step, 0)
        plsc.subcore_barrier()
        d0 = sid * drow
        pltpu.sync_copy(acc.at[pl.ds(d0, drow)],
                        out_hbm.at[pl.ds(cid * half + d0, drow)])

    return sk(msg, obj, zeros)


def _tc_message(HS, HR, QRW, Ws, Wr, WaT, Eb):
    E, D = HS.shape
    A = Ws.shape[1]
    sqc = float(MC) ** 0.5
    maxnorm = (1.0 - BALL_EPS) / sqc

    def body(hs_ref, hr_ref, qrw_ref, ws_ref, wr_ref, wat_ref, o_ref):
        hs = hs_ref[...]
        hr = hr_ref[...]
        pre = (jnp.dot(hs, ws_ref[...], preferred_element_type=jnp.float32)
               + jnp.dot(hr, wr_ref[...], preferred_element_type=jnp.float32)
               + qrw_ref[...])
        logit = jnp.sum(jnp.maximum(pre, 0.0) * wat_ref[...], axis=1,
                        keepdims=True)
        alpha = 1.0 / (1.0 + jnp.exp(-logit))

        def norm(x):
            return jnp.maximum(
                jnp.sqrt(jnp.sum(x * x, axis=-1, keepdims=True)), MN)

        def expmap0(u):
            un = norm(u)
            g = jnp.tanh(jnp.clip(sqc * un, -15.0, 15.0)) * u / (sqc * un)
            gn = norm(g)
            return jnp.where(gn > maxnorm, g / gn * maxnorm, g)

        hs_h = expmap0(hs)
        hr_h = expmap0(hr)
        x2 = jnp.sum(hs_h * hs_h, axis=-1, keepdims=True)
        y2 = jnp.sum(hr_h * hr_h, axis=-1, keepdims=True)
        xy = jnp.sum(hs_h * hr_h, axis=-1, keepdims=True)
        num = (1.0 + 2.0 * MC * xy + MC * y2) * hs_h + (1.0 - MC * x2) * hr_h
        den = 1.0 + 2.0 * MC * xy + MC * MC * x2 * y2
        z = num / jnp.maximum(den, MN)
        zn = norm(z)
        z = jnp.where(zn > maxnorm, z / zn * maxnorm, z)
        yn = norm(z)
        t = jnp.clip(sqc * yn, -1.0 + 1e-05, 1.0 - 1e-05)
        artanh = 0.5 * jnp.log((1.0 + t) / (1.0 - t))
        mess2 = z / yn / sqc * artanh
        o_ref[...] = mess2 * alpha

    grid = (E // Eb,)
    return pl.pallas_call(
        body,
        grid=grid,
        in_specs=[
            pl.BlockSpec((Eb, D), lambda i: (i, 0)),
            pl.BlockSpec((Eb, D), lambda i: (i, 0)),
            pl.BlockSpec((Eb, A), lambda i: (i, 0)),
            pl.BlockSpec((D, A), lambda i: (0, 0)),
            pl.BlockSpec((D, A), lambda i: (0, 0)),
            pl.BlockSpec((1, A), lambda i: (0, 0)),
        ],
        out_specs=pl.BlockSpec((Eb, D), lambda i: (i, 0)),
        out_shape=jax.ShapeDtypeStruct((E, D), jnp.float32),
    )(HS, HR, QRW, Ws, Wr, WaT)


def _tc_final(p, Wh):
    NP, D = p.shape

    def body(a_ref, w_ref, o_ref):
        o_ref[...] = jnp.dot(a_ref[...], w_ref[...],
                             preferred_element_type=jnp.float32)

    return pl.pallas_call(
        body,
        out_shape=jax.ShapeDtypeStruct((NP, D), jnp.float32),
    )(p, Wh)


def kernel(q_sub, q_rel, hidden, edges, n_node, old_nodes_new_idx,
           rela_embed, Ws, Wr, Wqr, bqr, Wa, Wh):
    N, D = hidden.shape
    E = edges.shape[0]
    A = Ws.shape[1]
    sub = edges[:, 4].astype(jnp.int32)
    rel = edges[:, 2].astype(jnp.int32)
    obj = edges[:, 5].astype(jnp.int32)
    ridx = edges[:, 0].astype(jnp.int32)

    Ap = 128
    pad = Ap - A
    qtabW = jnp.pad(rela_embed @ Wqr + bqr[None, :], ((0, 0), (0, pad)))
    Ws_p = jnp.pad(Ws, ((0, 0), (0, pad)))
    Wr_p = jnp.pad(Wr, ((0, 0), (0, pad)))
    WaT_p = jnp.pad(Wa.reshape(1, A), ((0, 0), (0, pad)))
    qsel = _sc_gather(qtabW, q_rel.astype(jnp.int32), K=32)
    QRW = _sc_gather(qsel, ridx, K=400)
    HS = _sc_gather(hidden, sub, K=400)
    HR = _sc_gather(rela_embed, rel, K=400)

    msg = _tc_message(HS, HR, QRW, Ws_p, Wr_p, WaT_p, Eb=2560)

    parts = _sc_scatter_add(msg, obj, half=5120, K=400)
    out = _tc_final(parts, Wh)
    return out[:N]


# speedup vs baseline: 2.9952x; 1.0473x over previous
import functools
import jax
import jax.numpy as jnp
from jax import lax
from jax.experimental import pallas as pl
from jax.experimental.pallas import tpu as pltpu
from jax.experimental.pallas import tpu_sc as plsc

MN = 1e-15
MC = 1e-06
BALL_EPS = 0.004


def _sc_gather(table, idx, K):
    M = idx.shape[0]
    C = table.shape[1]
    info = plsc.get_sparse_core_info()
    NC, NS = info.num_cores, info.num_subcores
    NW = NC * NS
    per_w = M // NW
    steps = per_w // K
    mesh = plsc.VectorSubcoreMesh(core_axis_name="c", subcore_axis_name="s")

    @functools.partial(
        pl.kernel, mesh=mesh,
        out_type=jax.ShapeDtypeStruct((M, C), jnp.float32),
        scratch_types=[
            pltpu.VMEM((K,), jnp.int32),
            pltpu.VMEM((K, C), jnp.float32),
            pltpu.SemaphoreType.DMA,
        ],
    )
    def gk(table_hbm, idx_hbm, out_hbm, idx_v, rows_v, sem):
        wid = lax.axis_index("s") * NC + lax.axis_index("c")
        base = wid * per_w

        def step(i, c):
            off = base + i * K
            pltpu.sync_copy(idx_hbm.at[pl.ds(off, K)], idx_v)
            pltpu.async_copy(table_hbm.at[idx_v], rows_v, sem).wait()
            pltpu.sync_copy(rows_v, out_hbm.at[pl.ds(off, K)])
            return c

        lax.fori_loop(0, steps, step, 0)

    return gk(table, idx)


def _sc_gather3(t0, i0, t1, i1, t2, i2, K):
    M = i0.shape[0]
    C = t0.shape[1]
    info = plsc.get_sparse_core_info()
    NC, NS = info.num_cores, info.num_subcores
    NW = NC * NS
    per_w = M // NW
    steps = per_w // K
    mesh = plsc.VectorSubcoreMesh(core_axis_name="c", subcore_axis_name="s")

    @functools.partial(
        pl.kernel, mesh=mesh,
        out_type=[jax.ShapeDtypeStruct((M, C), jnp.float32)] * 3,
        scratch_types=[
            pltpu.VMEM((K,), jnp.int32),
            pltpu.VMEM((K,), jnp.int32),
            pltpu.VMEM((K,), jnp.int32),
            pltpu.VMEM((K, C), jnp.float32),
            pltpu.VMEM((K, C), jnp.float32),
            pltpu.VMEM((K, C), jnp.float32),
            pltpu.SemaphoreType.DMA,
            pltpu.SemaphoreType.DMA,
            pltpu.SemaphoreType.DMA,
        ],
    )
    def gk(t0_h, i0_h, t1_h, i1_h, t2_h, i2_h, o0_h, o1_h, o2_h,
           x0, x1, x2, r0, r1, r2, s0, s1, s2):
        wid = lax.axis_index("s") * NC + lax.axis_index("c")
        base = wid * per_w

        def step(i, c):
            off = base + i * K
            pltpu.sync_copy(i0_h.at[pl.ds(off, K)], x0)
            d0 = pltpu.async_copy(t0_h.at[x0], r0, s0)
            pltpu.sync_copy(i1_h.at[pl.ds(off, K)], x1)
            d1 = pltpu.async_copy(t1_h.at[x1], r1, s1)
            pltpu.sync_copy(i2_h.at[pl.ds(off, K)], x2)
            d2 = pltpu.async_copy(t2_h.at[x2], r2, s2)
            d0.wait()
            pltpu.sync_copy(r0, o0_h.at[pl.ds(off, K)])
            d1.wait()
            pltpu.sync_copy(r1, o1_h.at[pl.ds(off, K)])
            d2.wait()
            pltpu.sync_copy(r2, o2_h.at[pl.ds(off, K)])
            return c

        lax.fori_loop(0, steps, step, 0)

    return gk(t0, i0, t1, i1, t2, i2)


def _sc_scatter_add(msg, obj, half, K):
    E, D = msg.shape
    info = plsc.get_sparse_core_info()
    NC, NS = info.num_cores, info.num_subcores
    per_s = E // NS
    steps = per_s // K
    AR = half + 128
    zrow = AR // NS
    drow = half // NS
    zeros = jnp.zeros((AR, D), jnp.float32)
    mesh = plsc.VectorSubcoreMesh(core_axis_name="c", subcore_axis_name="s")

    @functools.partial(
        pl.kernel, mesh=mesh,
        out_type=jax.ShapeDtypeStruct((NC * half, D), jnp.float32),
        scratch_types=[
            pltpu.VMEM((K,), jnp.int32),
            pltpu.VMEM((K, D), jnp.float32),
            pltpu.VMEM_SHARED((AR, D), jnp.float32),
        ],
    )
    def sk(msg_hbm, obj_hbm, z_hbm, out_hbm, idx_v, rows_v, acc):
        cid = lax.axis_index("c")
        sid = lax.axis_index("s")
        z0 = sid * zrow
        pltpu.sync_copy(z_hbm.at[pl.ds(z0, zrow)], acc.at[pl.ds(z0, zrow)])
        plsc.subcore_barrier()
        lo = cid * half
        base = sid * per_s

        def step(i, c):
            off = base + i * K
            pltpu.sync_copy(obj_hbm.at[pl.ds(off, K)], idx_v)
            pltpu.sync_copy(msg_hbm.at[pl.ds(off, K)], rows_v)

            def remap(j, c2):
                v = idx_v[pl.ds(j * 16, 16)]
                loc = v - lo
                ok = (loc >= 0) & (loc < half)
                idx_v[pl.ds(j * 16, 16)] = jnp.where(ok, loc, half)
                return c2

            lax.fori_loop(0, K // 16, remap, 0)
            pltpu.sync_copy(rows_v, acc.at[idx_v], add=True)
            return c

        lax.fori_loop(0, steps, step, 0)
        plsc.subcore_barrier()
        d0 = sid * drow
        pltpu.sync_copy(acc.at[pl.ds(d0, drow)],
                        out_hbm.at[pl.ds(cid * half + d0, drow)])

    return sk(msg, obj, zeros)


def _tc_message(HS, HR, QRW, Ws, Wr, WaT, Eb):
    E, D = HS.shape
    A = Ws.shape[1]
    sqc = float(MC) ** 0.5
    maxnorm = (1.0 - BALL_EPS) / sqc

    def body(hs_ref, hr_ref, qrw_ref, ws_ref, wr_ref, wat_ref, o_ref):
        hs = hs_ref[...]
        hr = hr_ref[...]
        pre = (jnp.dot(hs, ws_ref[...], preferred_element_type=jnp.float32)
               + jnp.dot(hr, wr_ref[...], preferred_element_type=jnp.float32)
               + qrw_ref[...])
        logit = jnp.sum(jnp.maximum(pre, 0.0) * wat_ref[...], axis=1,
                        keepdims=True)
        alpha = 1.0 / (1.0 + jnp.exp(-logit))

        def norm(x):
            return jnp.maximum(
                jnp.sqrt(jnp.sum(x * x, axis=-1, keepdims=True)), MN)

        def expmap0(u):
            un = norm(u)
            g = jnp.tanh(jnp.clip(sqc * un, -15.0, 15.0)) * u / (sqc * un)
            gn = norm(g)
            return jnp.where(gn > maxnorm, g / gn * maxnorm, g)

        hs_h = expmap0(hs)
        hr_h = expmap0(hr)
        x2 = jnp.sum(hs_h * hs_h, axis=-1, keepdims=True)
        y2 = jnp.sum(hr_h * hr_h, axis=-1, keepdims=True)
        xy = jnp.sum(hs_h * hr_h, axis=-1, keepdims=True)
        num = (1.0 + 2.0 * MC * xy + MC * y2) * hs_h + (1.0 - MC * x2) * hr_h
        den = 1.0 + 2.0 * MC * xy + MC * MC * x2 * y2
        z = num / jnp.maximum(den, MN)
        zn = norm(z)
        z = jnp.where(zn > maxnorm, z / zn * maxnorm, z)
        yn = norm(z)
        t = jnp.clip(sqc * yn, -1.0 + 1e-05, 1.0 - 1e-05)
        artanh = 0.5 * jnp.log((1.0 + t) / (1.0 - t))
        mess2 = z / yn / sqc * artanh
        o_ref[...] = mess2 * alpha

    grid = (E // Eb,)
    return pl.pallas_call(
        body,
        grid=grid,
        in_specs=[
            pl.BlockSpec((Eb, D), lambda i: (i, 0)),
            pl.BlockSpec((Eb, D), lambda i: (i, 0)),
            pl.BlockSpec((Eb, A), lambda i: (i, 0)),
            pl.BlockSpec((D, A), lambda i: (0, 0)),
            pl.BlockSpec((D, A), lambda i: (0, 0)),
            pl.BlockSpec((1, A), lambda i: (0, 0)),
        ],
        out_specs=pl.BlockSpec((Eb, D), lambda i: (i, 0)),
        out_shape=jax.ShapeDtypeStruct((E, D), jnp.float32),
    )(HS, HR, QRW, Ws, Wr, WaT)


def _tc_final(p, Wh):
    NP, D = p.shape

    def body(a_ref, w_ref, o_ref):
        o_ref[...] = jnp.dot(a_ref[...], w_ref[...],
                             preferred_element_type=jnp.float32)

    return pl.pallas_call(
        body,
        out_shape=jax.ShapeDtypeStruct((NP, D), jnp.float32),
    )(p, Wh)


def kernel(q_sub, q_rel, hidden, edges, n_node, old_nodes_new_idx,
           rela_embed, Ws, Wr, Wqr, bqr, Wa, Wh):
    N, D = hidden.shape
    E = edges.shape[0]
    A = Ws.shape[1]
    sub = edges[:, 4].astype(jnp.int32)
    rel = edges[:, 2].astype(jnp.int32)
    obj = edges[:, 5].astype(jnp.int32)
    ridx = edges[:, 0].astype(jnp.int32)

    Ap = 128
    pad = Ap - A
    qtabW = jnp.pad(rela_embed @ Wqr + bqr[None, :], ((0, 0), (0, pad)))
    Ws_p = jnp.pad(Ws, ((0, 0), (0, pad)))
    Wr_p = jnp.pad(Wr, ((0, 0), (0, pad)))
    WaT_p = jnp.pad(Wa.reshape(1, A), ((0, 0), (0, pad)))
    qsel = _sc_gather(qtabW, q_rel.astype(jnp.int32), K=32)
    QRW, HS, HR = _sc_gather3(qsel, ridx, hidden, sub,
                              rela_embed, rel, K=200)

    msg = _tc_message(HS, HR, QRW, Ws_p, Wr_p, WaT_p, Eb=2560)

    parts = _sc_scatter_add(msg, obj, half=5120, K=400)
    out = _tc_final(parts, Wh)
    return out[:N]
